# Initial kernel scaffold; baseline (speedup 1.0000x reference)
import jax
import jax.numpy as jnp
from jax.experimental import pallas as pl


def kernel(user_emb, item_emb, adj_val, users, pos_items, neg_items, adj_row, adj_col, mask, norm_adj):
    raise NotImplementedError("write your pallas kernel here")


# speedup vs baseline: 3.5719x; 3.5719x over previous
import functools

import jax
import jax.numpy as jnp
from jax import lax
from jax.experimental import pallas as pl
from jax.experimental.pallas import tpu as pltpu
from jax.experimental.pallas import tpu_sc as plsc

N_USER = 25000
N_ITEM = 25000
N = N_USER + N_ITEM
E = 800000
EMB = 64
HALF = EMB // 2
BATCH = 1024
N_NEG = 4096
NOUT = BATCH + BATCH + N_NEG

K = 128
NCHUNK = E // K
NS = 16
CHUNKS_PER_SUB = -(-NCHUNK // NS)
ZROWS = 125
ZREP = (N // NS) // ZROWS
OUT_ROWS = NOUT // K
OUT_PER_SUB = OUT_ROWS // NS


def _half_pass(ego_ref, out_ref, row2, col2, val2, need2, acc, colbuf,
               valbuf, rowbuf, gbuf, zbuf, ebuf, abuf):
    s = lax.axis_index("s")

    zero16 = jnp.zeros((16,), jnp.float32)

    @pl.loop(0, ZROWS)
    def _(i):
        zbuf[i, pl.ds(0, 16)] = zero16
        zbuf[i, pl.ds(16, 16)] = zero16

    @pl.loop(0, ZREP)
    def _(j):
        pltpu.sync_copy(zbuf, acc.at[pl.ds(s * (N // NS) + j * ZROWS, ZROWS)])

    plsc.subcore_barrier()

    @pl.loop(0, CHUNKS_PER_SUB)
    def _(i):
        r = i * NS + s

        @pl.when(r < NCHUNK)
        def _():
            pltpu.sync_copy(col2.at[r], colbuf)
            pltpu.sync_copy(val2.at[r], valbuf)
            pltpu.sync_copy(row2.at[r], rowbuf)
            pltpu.sync_copy(ego_ref.at[colbuf], gbuf)

            @pl.loop(0, K, step=16)
            def _(c0):
                vals = valbuf[pl.ds(c0, 16)]
                for i in range(16):
                    vi = vals.at[jnp.full((16,), i, jnp.int32)].get(
                        mode="promise_in_bounds")
                    k = c0 + i
                    gbuf[k, pl.ds(0, 16)] = gbuf[k, pl.ds(0, 16)] * vi
                    gbuf[k, pl.ds(16, 16)] = gbuf[k, pl.ds(16, 16)] * vi

            pltpu.sync_copy(gbuf, acc.at[rowbuf], add=True)

    plsc.subcore_barrier()

    @pl.loop(0, OUT_PER_SUB)
    def _(t):
        r = s * OUT_PER_SUB + t
        pltpu.sync_copy(need2.at[r], colbuf)
        pltpu.sync_copy(ego_ref.at[colbuf], ebuf)
        pltpu.sync_copy(acc.at[colbuf], abuf)

        third = jnp.float32(1.0 / 3.0)

        @pl.loop(0, K)
        def _(k):
            ebuf[k, pl.ds(0, 16)] = (
                ebuf[k, pl.ds(0, 16)] + 2.0 * abuf[k, pl.ds(0, 16)]) * third
            ebuf[k, pl.ds(16, 16)] = (
                ebuf[k, pl.ds(16, 16)] + 2.0 * abuf[k, pl.ds(16, 16)]) * third

        pltpu.sync_copy(ebuf, out_ref.at[pl.ds(r * K, K)])


def _sc_kernel_body(ego_lo, ego_hi, row2, col2, val2, need2, out_lo, out_hi,
                    acc, colbuf, valbuf, rowbuf, gbuf, zbuf, ebuf, abuf):
    c = lax.axis_index("c")

    @pl.when(c == 0)
    def _():
        _half_pass(ego_lo, out_lo, row2, col2, val2, need2, acc, colbuf,
                   valbuf, rowbuf, gbuf, zbuf, ebuf, abuf)

    @pl.when(c == 1)
    def _():
        _half_pass(ego_hi, out_hi, row2, col2, val2, need2, acc, colbuf,
                   valbuf, rowbuf, gbuf, zbuf, ebuf, abuf)


@jax.jit
def _run(ego_lo, ego_hi, row2, col2, val2, need2):
    mesh = plsc.VectorSubcoreMesh(core_axis_name="c", subcore_axis_name="s")
    f32 = jnp.float32
    fn = pl.kernel(
        _sc_kernel_body,
        out_type=(
            jax.ShapeDtypeStruct((NOUT, HALF), f32),
            jax.ShapeDtypeStruct((NOUT, HALF), f32),
        ),
        mesh=mesh,
        compiler_params=pltpu.CompilerParams(use_tc_tiling_on_sc=False),
        scratch_types=[
            pltpu.VMEM_SHARED((N, HALF), f32),
            pltpu.VMEM((K,), jnp.int32),
            pltpu.VMEM((K,), f32),
            pltpu.VMEM((K,), jnp.int32),
            pltpu.VMEM((K, HALF), f32),
            pltpu.VMEM((ZROWS, HALF), f32),
            pltpu.VMEM((K, HALF), f32),
            pltpu.VMEM((K, HALF), f32),
        ],
    )
    return fn(ego_lo, ego_hi, row2, col2, val2, need2)


def kernel(user_emb, item_emb, adj_val, users, pos_items, neg_items,
           adj_row, adj_col, mask, norm_adj):
    ego = jnp.concatenate([user_emb, item_emb], axis=0)
    ego_lo = ego[:, :HALF]
    ego_hi = ego[:, HALF:]
    row2 = adj_row.reshape(NCHUNK, K)
    col2 = adj_col.reshape(NCHUNK, K)
    val2 = adj_val.reshape(NCHUNK, K)
    need = jnp.concatenate(
        [users, pos_items + N_USER, neg_items + N_USER]).astype(jnp.int32)
    need2 = need.reshape(OUT_ROWS, K)
    out_lo, out_hi = _run(ego_lo, ego_hi, row2, col2, val2, need2)
    light = jnp.concatenate([out_lo, out_hi], axis=1)
    return (light[:BATCH], light[BATCH:2 * BATCH], light[2 * BATCH:])


# speedup vs baseline: 9.6931x; 2.7137x over previous
import dataclasses

import jax
import jax.numpy as jnp
from jax import lax
from jax.experimental import pallas as pl
from jax.experimental.pallas import tpu as pltpu
from jax.experimental.pallas import tpu_sc as plsc

N_USER = 25000
N_ITEM = 25000
N = N_USER + N_ITEM
E = 800000
EMB = 64
HALF = EMB // 2
BATCH = 1024
N_NEG = 4096
NOUT = BATCH + BATCH + N_NEG

NS = 16
SB = 1024
NSB = 49
E2 = NS * SB * NSB
K = 128
STG = 1280
ACC_ROWS = 6400
ZROWS = 100
ZREP = (ACC_ROWS // NS) // ZROWS
OUT_PER_SUB = (NOUT // K) // NS


def _half_pass(ego_ref, out_ref, rowf, colf, valf, need, acc, remap, needbuf,
               rbuf, cbuf, vbuf, scol, sval, srow, fcol, fval, frow, ridx,
               gbuf, zbuf, ebuf, abuf):
    s = lax.axis_index("s")
    zero16i = jnp.zeros((16,), jnp.int32)
    zero16f = jnp.zeros((16,), jnp.float32)
    neg16i = jnp.full((16,), -1, jnp.int32)
    iota16 = lax.iota(jnp.int32, 16)

    @pl.loop(0, N // 16)
    def _(i):
        remap[pl.ds(i * 16, 16)] = neg16i

    pltpu.sync_copy(need, needbuf)

    @pl.loop(0, NOUT // 16)
    def _(i):
        idx16 = needbuf[pl.ds(i * 16, 16)]
        plsc.store_scatter(remap, [idx16], i * 16 + iota16)

    @pl.loop(0, ZROWS)
    def _(i):
        zbuf[i, pl.ds(0, 16)] = zero16f
        zbuf[i, pl.ds(16, 16)] = zero16f

    @pl.loop(0, ZREP)
    def _(j):
        pltpu.sync_copy(
            zbuf, acc.at[pl.ds(s * (ACC_ROWS // NS) + j * ZROWS, ZROWS)])

    plsc.subcore_barrier()

    @pl.loop(0, NSB)
    def _(b):
        base = (s * NSB + b) * SB
        pltpu.sync_copy(rowf.at[pl.ds(base, SB)], rbuf)
        pltpu.sync_copy(colf.at[pl.ds(base, SB)], cbuf)
        pltpu.sync_copy(valf.at[pl.ds(base, SB)], vbuf)

        def grp(g, pos):
            r16 = rbuf[pl.ds(g * 16, 16)]
            m16 = plsc.load_gather(remap, [r16])
            pred = m16 >= 0
            c16 = cbuf[pl.ds(g * 16, 16)]
            v16 = vbuf[pl.ds(g * 16, 16)]
            plsc.store_compressed(scol.at[pl.ds(pos, 16)], c16, mask=pred)
            plsc.store_compressed(sval.at[pl.ds(pos, 16)], v16, mask=pred)
            plsc.store_compressed(srow.at[pl.ds(pos, 16)], m16, mask=pred)
            cnt = plsc.all_reduce_population_count(pred)
            return pos + cnt[0]

        pos = lax.fori_loop(0, SB // 16, grp, jnp.int32(0), unroll=2)

        for i in range(8):
            scol[pl.ds(pos + i * 16, 16)] = zero16i
            sval[pl.ds(pos + i * 16, 16)] = zero16f
            srow[pl.ds(pos + i * 16, 16)] = zero16i

        def fire_cond(j):
            return j * K < pos

        def fire_body(j):
            fb = j * K
            for i in range(8):
                fcol[pl.ds(i * 16, 16)] = scol[pl.ds(fb + i * 16, 16)]
                fval[pl.ds(i * 16, 16)] = sval[pl.ds(fb + i * 16, 16)]
                frow[pl.ds(i * 16, 16)] = srow[pl.ds(fb + i * 16, 16)]
            pltpu.sync_copy(ego_ref.at[fcol], gbuf)

            @pl.loop(0, K, step=16)
            def _(c0):
                vals = fval[pl.ds(c0, 16)]
                for i in range(16):
                    vi = vals.at[jnp.full((16,), i, jnp.int32)].get(
                        mode="promise_in_bounds")
                    k = c0 + i
                    gbuf[k, pl.ds(0, 16)] = gbuf[k, pl.ds(0, 16)] * vi
                    gbuf[k, pl.ds(16, 16)] = gbuf[k, pl.ds(16, 16)] * vi

            pltpu.sync_copy(gbuf, acc.at[frow], add=True)
            return j + 1

        lax.while_loop(fire_cond, fire_body, jnp.int32(0))

    plsc.subcore_barrier()

    @pl.loop(0, OUT_PER_SUB)
    def _(t):
        r = s * OUT_PER_SUB + t
        nidx = needbuf.at[pl.ds(r * K, K)]
        for i in range(8):
            n16 = needbuf[pl.ds(r * K + i * 16, 16)]
            ridx[pl.ds(i * 16, 16)] = plsc.load_gather(remap, [n16])
        pltpu.sync_copy(ego_ref.at[nidx], ebuf)
        pltpu.sync_copy(acc.at[ridx], abuf)

        third = jnp.float32(1.0 / 3.0)

        @pl.loop(0, K)
        def _(k):
            ebuf[k, pl.ds(0, 16)] = (
                ebuf[k, pl.ds(0, 16)] + 2.0 * abuf[k, pl.ds(0, 16)]) * third
            ebuf[k, pl.ds(16, 16)] = (
                ebuf[k, pl.ds(16, 16)] + 2.0 * abuf[k, pl.ds(16, 16)]) * third

        pltpu.sync_copy(ebuf, out_ref.at[pl.ds(r * K, K)])


def _sc_kernel_body(ego_lo, ego_hi, rowf, colf, valf, need, out_lo, out_hi,
                    acc, remap, needbuf, rbuf, cbuf, vbuf, scol, sval, srow,
                    fcol, fval, frow, ridx, gbuf, zbuf, ebuf, abuf):
    c = lax.axis_index("c")

    @pl.when(c == 0)
    def _():
        _half_pass(ego_lo, out_lo, rowf, colf, valf, need, acc, remap,
                   needbuf, rbuf, cbuf, vbuf, scol, sval, srow, fcol, fval,
                   frow, ridx, gbuf, zbuf, ebuf, abuf)

    @pl.when(c == 1)
    def _():
        _half_pass(ego_hi, out_hi, rowf, colf, valf, need, acc, remap,
                   needbuf, rbuf, cbuf, vbuf, scol, sval, srow, fcol, fval,
                   frow, ridx, gbuf, zbuf, ebuf, abuf)


def _compiler_params():
    cp = pltpu.CompilerParams(use_tc_tiling_on_sc=False)
    if "needs_layout_passes" in pltpu.CompilerParams.__dataclass_fields__:
        cp = dataclasses.replace(cp, needs_layout_passes=False)
    return cp


@jax.jit
def _run(ego_lo, ego_hi, rowf, colf, valf, need):
    mesh = plsc.VectorSubcoreMesh(core_axis_name="c", subcore_axis_name="s")
    f32 = jnp.float32
    i32 = jnp.int32
    fn = pl.kernel(
        _sc_kernel_body,
        out_type=(
            jax.ShapeDtypeStruct((NOUT, HALF), f32),
            jax.ShapeDtypeStruct((NOUT, HALF), f32),
        ),
        mesh=mesh,
        compiler_params=_compiler_params(),
        scratch_types=[
            pltpu.VMEM_SHARED((ACC_ROWS, HALF), f32),
            pltpu.VMEM((N,), i32),
            pltpu.VMEM((NOUT,), i32),
            pltpu.VMEM((SB,), i32),
            pltpu.VMEM((SB,), i32),
            pltpu.VMEM((SB,), f32),
            pltpu.VMEM((STG,), i32),
            pltpu.VMEM((STG,), f32),
            pltpu.VMEM((STG,), i32),
            pltpu.VMEM((K,), i32),
            pltpu.VMEM((K,), f32),
            pltpu.VMEM((K,), i32),
            pltpu.VMEM((K,), i32),
            pltpu.VMEM((K, HALF), f32),
            pltpu.VMEM((ZROWS, HALF), f32),
            pltpu.VMEM((K, HALF), f32),
            pltpu.VMEM((K, HALF), f32),
        ],
    )
    return fn(ego_lo, ego_hi, rowf, colf, valf, need)


def kernel(user_emb, item_emb, adj_val, users, pos_items, neg_items,
           adj_row, adj_col, mask, norm_adj):
    ego = jnp.concatenate([user_emb, item_emb], axis=0)
    ego_lo = ego[:, :HALF]
    ego_hi = ego[:, HALF:]
    pad = E2 - E
    rowf = jnp.concatenate([adj_row, jnp.zeros((pad,), jnp.int32)])
    colf = jnp.concatenate([adj_col, jnp.zeros((pad,), jnp.int32)])
    valf = jnp.concatenate([adj_val, jnp.zeros((pad,), jnp.float32)])
    need = jnp.concatenate(
        [users, pos_items + N_USER, neg_items + N_USER]).astype(jnp.int32)
    out_lo, out_hi = _run(ego_lo, ego_hi, rowf, colf, valf, need)
    light = jnp.concatenate([out_lo, out_hi], axis=1)
    return (light[:BATCH], light[BATCH:2 * BATCH], light[2 * BATCH:])


# speedup vs baseline: 12.4194x; 1.2813x over previous
import dataclasses

import jax
import jax.numpy as jnp
from jax import lax
from jax.experimental import pallas as pl
from jax.experimental.pallas import tpu as pltpu
from jax.experimental.pallas import tpu_sc as plsc

N_USER = 25000
N_ITEM = 25000
N = N_USER + N_ITEM
E = 800000
EMB = 64
HALF = EMB // 2
BATCH = 1024
N_NEG = 4096
NOUT = BATCH + BATCH + N_NEG

NS = 16
SB = 4096
NSB = 13
E2 = NS * SB * NSB
K = 128
STG = SB + 2 * K
ACC_ROWS = 6400
ZROWS = 100
ZREP = (ACC_ROWS // NS) // ZROWS
OUT_PER_SUB = (NOUT // K) // NS


def _half_pass(ego_ref, out_ref, edges3, need, acc, remap, needbuf,
               eb0, eb1, scol, sval, srow, fcol, fval, frow, ridx,
               gbuf, zbuf, ebuf, abuf, sem0, sem1):
    s = lax.axis_index("s")
    zero16i = jnp.zeros((16,), jnp.int32)
    zero16f = jnp.zeros((16,), jnp.float32)
    neg16i = jnp.full((16,), -1, jnp.int32)
    iota16 = lax.iota(jnp.int32, 16)

    @pl.loop(0, N // 16)
    def _(i):
        remap[pl.ds(i * 16, 16)] = neg16i

    pltpu.sync_copy(need, needbuf)

    @pl.loop(0, NOUT // 16)
    def _(i):
        idx16 = needbuf[pl.ds(i * 16, 16)]
        plsc.store_scatter(remap, [idx16], i * 16 + iota16)

    @pl.loop(0, ZROWS)
    def _(i):
        zbuf[i, pl.ds(0, 16)] = zero16f
        zbuf[i, pl.ds(16, 16)] = zero16f

    @pl.loop(0, ZREP)
    def _(j):
        pltpu.sync_copy(
            zbuf, acc.at[pl.ds(s * (ACC_ROWS // NS) + j * ZROWS, ZROWS)])

    plsc.subcore_barrier()

    def process_sb(buf):
        def grp(g, pos):
            r16 = buf[0, pl.ds(g * 16, 16)]
            m16 = plsc.load_gather(remap, [r16])
            pred = m16 >= 0
            c16 = buf[1, pl.ds(g * 16, 16)]
            v16 = plsc.bitcast(buf[2, pl.ds(g * 16, 16)], jnp.float32)
            plsc.store_compressed(scol.at[pl.ds(pos, 16)], c16, mask=pred)
            plsc.store_compressed(sval.at[pl.ds(pos, 16)], v16, mask=pred)
            plsc.store_compressed(srow.at[pl.ds(pos, 16)], m16, mask=pred)
            cnt = plsc.all_reduce_population_count(pred)
            return pos + cnt[0]

        pos = lax.fori_loop(0, SB // 16, grp, jnp.int32(0), unroll=2)

        for i in range(8):
            scol[pl.ds(pos + i * 16, 16)] = zero16i
            sval[pl.ds(pos + i * 16, 16)] = zero16f
            srow[pl.ds(pos + i * 16, 16)] = zero16i

        def fire_cond(j):
            return j * K < pos

        def fire_body(j):
            fb = j * K
            for i in range(8):
                fcol[pl.ds(i * 16, 16)] = scol[pl.ds(fb + i * 16, 16)]
                fval[pl.ds(i * 16, 16)] = sval[pl.ds(fb + i * 16, 16)]
                frow[pl.ds(i * 16, 16)] = srow[pl.ds(fb + i * 16, 16)]
            pltpu.sync_copy(ego_ref.at[fcol], gbuf)

            @pl.loop(0, K, step=16)
            def _(c0):
                vals = fval[pl.ds(c0, 16)]
                for i in range(16):
                    vi = vals.at[jnp.full((16,), i, jnp.int32)].get(
                        mode="promise_in_bounds")
                    k = c0 + i
                    gbuf[k, pl.ds(0, 16)] = gbuf[k, pl.ds(0, 16)] * vi
                    gbuf[k, pl.ds(16, 16)] = gbuf[k, pl.ds(16, 16)] * vi

            pltpu.sync_copy(gbuf, acc.at[frow], add=True)
            return j + 1

        lax.while_loop(fire_cond, fire_body, jnp.int32(0))

    bufs = (eb0, eb1)
    sems = (sem0, sem1)
    handles = [None, None]
    handles[0] = pltpu.async_copy(edges3.at[s * NSB], eb0, sem0)
    for b in range(NSB):
        p = b % 2
        handles[p].wait()
        if b + 1 < NSB:
            handles[1 - p] = pltpu.async_copy(
                edges3.at[s * NSB + (b + 1)], bufs[1 - p], sems[1 - p])
        process_sb(bufs[p])

    plsc.subcore_barrier()

    @pl.loop(0, OUT_PER_SUB)
    def _(t):
        r = s * OUT_PER_SUB + t
        nidx = needbuf.at[pl.ds(r * K, K)]
        for i in range(8):
            n16 = needbuf[pl.ds(r * K + i * 16, 16)]
            ridx[pl.ds(i * 16, 16)] = plsc.load_gather(remap, [n16])
        pltpu.sync_copy(ego_ref.at[nidx], ebuf)
        pltpu.sync_copy(acc.at[ridx], abuf)

        third = jnp.float32(1.0 / 3.0)

        @pl.loop(0, K)
        def _(k):
            ebuf[k, pl.ds(0, 16)] = (
                ebuf[k, pl.ds(0, 16)] + 2.0 * abuf[k, pl.ds(0, 16)]) * third
            ebuf[k, pl.ds(16, 16)] = (
                ebuf[k, pl.ds(16, 16)] + 2.0 * abuf[k, pl.ds(16, 16)]) * third

        pltpu.sync_copy(ebuf, out_ref.at[pl.ds(r * K, K)])


def _sc_kernel_body(ego_lo, ego_hi, edges3, need, out_lo, out_hi,
                    acc, remap, needbuf, eb0, eb1, scol, sval, srow,
                    fcol, fval, frow, ridx, gbuf, zbuf, ebuf, abuf,
                    sem0, sem1):
    c = lax.axis_index("c")

    @pl.when(c == 0)
    def _():
        _half_pass(ego_lo, out_lo, edges3, need, acc, remap, needbuf,
                   eb0, eb1, scol, sval, srow, fcol, fval, frow, ridx,
                   gbuf, zbuf, ebuf, abuf, sem0, sem1)

    @pl.when(c == 1)
    def _():
        _half_pass(ego_hi, out_hi, edges3, need, acc, remap, needbuf,
                   eb0, eb1, scol, sval, srow, fcol, fval, frow, ridx,
                   gbuf, zbuf, ebuf, abuf, sem0, sem1)


def _compiler_params():
    cp = pltpu.CompilerParams(use_tc_tiling_on_sc=False)
    if "needs_layout_passes" in pltpu.CompilerParams.__dataclass_fields__:
        cp = dataclasses.replace(cp, needs_layout_passes=False)
    return cp


@jax.jit
def _run(ego_lo, ego_hi, edges3, need):
    mesh = plsc.VectorSubcoreMesh(core_axis_name="c", subcore_axis_name="s")
    f32 = jnp.float32
    i32 = jnp.int32
    fn = pl.kernel(
        _sc_kernel_body,
        out_type=(
            jax.ShapeDtypeStruct((NOUT, HALF), f32),
            jax.ShapeDtypeStruct((NOUT, HALF), f32),
        ),
        mesh=mesh,
        compiler_params=_compiler_params(),
        scratch_types=[
            pltpu.VMEM_SHARED((ACC_ROWS, HALF), f32),
            pltpu.VMEM((N,), i32),
            pltpu.VMEM((NOUT,), i32),
            pltpu.VMEM((3, SB), i32),
            pltpu.VMEM((3, SB), i32),
            pltpu.VMEM((STG,), i32),
            pltpu.VMEM((STG,), f32),
            pltpu.VMEM((STG,), i32),
            pltpu.VMEM((K,), i32),
            pltpu.VMEM((K,), f32),
            pltpu.VMEM((K,), i32),
            pltpu.VMEM((K,), i32),
            pltpu.VMEM((K, HALF), f32),
            pltpu.VMEM((ZROWS, HALF), f32),
            pltpu.VMEM((K, HALF), f32),
            pltpu.VMEM((K, HALF), f32),
            pltpu.SemaphoreType.DMA,
            pltpu.SemaphoreType.DMA,
        ],
    )
    return fn(ego_lo, ego_hi, edges3, need)


def kernel(user_emb, item_emb, adj_val, users, pos_items, neg_items,
           adj_row, adj_col, mask, norm_adj):
    ego = jnp.concatenate([user_emb, item_emb], axis=0)
    ego_lo = ego[:, :HALF]
    ego_hi = ego[:, HALF:]
    pad = E2 - E
    zi = jnp.zeros((pad,), jnp.int32)
    rowf = jnp.concatenate([adj_row, zi])
    colf = jnp.concatenate([adj_col, zi])
    valf = jnp.concatenate([adj_val, jnp.zeros((pad,), jnp.float32)])
    vbits = lax.bitcast_convert_type(valf, jnp.int32)
    edges3 = (jnp.stack([rowf, colf, vbits])
              .reshape(3, NS * NSB, SB).transpose(1, 0, 2))
    need = jnp.concatenate(
        [users, pos_items + N_USER, neg_items + N_USER]).astype(jnp.int32)
    out_lo, out_hi = _run(ego_lo, ego_hi, edges3, need)
    light = jnp.concatenate([out_lo, out_hi], axis=1)
    return (light[:BATCH], light[BATCH:2 * BATCH], light[2 * BATCH:])
